# baseline (speedup 1.0000x reference)
import functools
import jax
import jax.numpy as jnp
import numpy as np
from itertools import combinations
from jax import lax
from jax.experimental import pallas as pl
from jax.experimental.pallas import tpu as pltpu
from jax.experimental.pallas import tpu_sc as plsc

_NF = 26
_ED = 64
_B = 4096
_FP = 32
_Q = _NF * _FP
_L = 16
_NW = 32
_NG = _B // _L
_GPW = _NG // _NW
_QL = _Q * _L
_XW = _NF * _ED * _L

_pairs = list(combinations(range(_NF), 2))
_COLS_NP = np.array([p[0] for p in _pairs], dtype=np.int32)
_ROWS_NP = np.array([p[1] for p in _pairs], dtype=np.int32)
_QIDX_NP = _ROWS_NP * _FP + _COLS_NP


def _sc_moments_body(xg_hbm, inter_hbm, s1_hbm, s2_hbm, xbuf, interbuf, s1buf, s2buf, pacc):
    wid = lax.axis_index("s") * 2 + lax.axis_index("c")
    zv = jnp.zeros((_L,), jnp.float32)

    def zero_body(q, _):
        s1buf[pl.ds(q * _L, _L)] = zv
        s2buf[pl.ds(q * _L, _L)] = zv
        return 0

    lax.fori_loop(0, _Q, zero_body, 0, unroll=False)

    def group_body(gl, _):
        grp = wid * _GPW + gl
        pltpu.sync_copy(xg_hbm.at[grp], xbuf)

        def j_body(j, _):
            for dc in range(_ED // _L):
                xj = [xbuf[pl.ds((j * _ED + dc * _L + t) * _L, _L)] for t in range(_L)]

                def i_body(i, _):
                    s = xbuf[pl.ds((i * _ED + dc * _L) * _L, _L)] * xj[0]
                    for t in range(1, _L):
                        s = s + xbuf[pl.ds((i * _ED + dc * _L + t) * _L, _L)] * xj[t]
                    if dc == 0:
                        pacc[pl.ds(i * _L, _L)] = s
                    else:
                        pacc[pl.ds(i * _L, _L)] = pacc[pl.ds(i * _L, _L)] + s
                    return 0

                lax.fori_loop(0, j, i_body, 0, unroll=False)

            def stat_body(i, _):
                v = pacc[pl.ds(i * _L, _L)]
                q = j * _FP + i
                interbuf[pl.ds(q * _L, _L)] = v
                s1buf[pl.ds(q * _L, _L)] = s1buf[pl.ds(q * _L, _L)] + v
                s2buf[pl.ds(q * _L, _L)] = s2buf[pl.ds(q * _L, _L)] + v * v
                return 0

            lax.fori_loop(0, j, stat_body, 0, unroll=False)
            return 0

        lax.fori_loop(1, _NF, j_body, 0, unroll=False)
        pltpu.sync_copy(interbuf, inter_hbm.at[grp])
        return 0

    lax.fori_loop(0, _GPW, group_body, 0, unroll=False)
    pltpu.sync_copy(s1buf, s1_hbm.at[wid])
    pltpu.sync_copy(s2buf, s2_hbm.at[wid])


def _chunk_total(t):
    for step in (8, 4, 2, 1):
        t = t + jnp.roll(t, -step, axis=1)
    lane = jax.lax.broadcasted_iota(jnp.int32, (1, _QL), 1)
    for step in (1, 2, 4, 8):
        t = jnp.where(lane % (2 * step) < step, t, jnp.roll(t, step, axis=1))
    return t


def _finalize_kernel(s1p_ref, s2p_ref, wm_ref, gm_ref, bm_ref, alphav_ref, cvec_ref):
    s1 = _chunk_total(jnp.sum(s1p_ref[...], axis=0, keepdims=True))
    s2 = _chunk_total(jnp.sum(s2p_ref[...], axis=0, keepdims=True))
    mean = s1 * (1.0 / _B)
    var = s2 * (1.0 / _B) - mean * mean
    rstd = jax.lax.rsqrt(var + 1e-5)
    wm = wm_ref[...]
    gm = gm_ref[...]
    bm = bm_ref[...]
    alpha = wm * gm * rstd
    cval = jnp.sum(wm * (bm - gm * mean * rstd)) * (1.0 / _L)
    alphav_ref[...] = alpha
    cvec_ref[...] = jnp.broadcast_to(cval.reshape(1, 1), (1, _L))


def _sc_out_body(inter_hbm, alphav_hbm, cvec_hbm, out_hbm, ibuf, abuf, cbuf, obuf):
    wid = lax.axis_index("s") * 2 + lax.axis_index("c")
    pltpu.sync_copy(alphav_hbm, abuf)
    pltpu.sync_copy(cvec_hbm, cbuf)

    def group_body(gl, _):
        grp = wid * _GPW + gl
        pltpu.sync_copy(inter_hbm.at[grp], ibuf)
        obuf[...] = cbuf[...]

        def j_body(j, _):
            def i_body(i, _):
                q = j * _FP + i
                obuf[...] = obuf[...] + abuf[pl.ds(q * _L, _L)] * ibuf[pl.ds(q * _L, _L)]
                return 0

            lax.fori_loop(0, j, i_body, 0, unroll=False)
            return 0

        lax.fori_loop(1, _NF, j_body, 0, unroll=False)
        pltpu.sync_copy(obuf, out_hbm.at[pl.ds(grp * _L, _L)])
        return 0

    lax.fori_loop(0, _GPW, group_body, 0, unroll=False)


def kernel(x, bn_gamma, bn_beta, edge_weights):
    xg = jnp.transpose(x.reshape(_NG, _L, _NF, _ED), (0, 2, 3, 1)).reshape(_NG, _XW)

    def scat(v):
        m = jnp.zeros((_Q,), jnp.float32).at[_QIDX_NP].set(v)
        return jnp.broadcast_to(m.reshape(_Q, 1), (_Q, _L)).reshape(1, _QL)

    wm = scat(edge_weights)
    gm = scat(bn_gamma)
    bm = scat(bn_beta)

    mesh = plsc.VectorSubcoreMesh(core_axis_name="c", subcore_axis_name="s")

    moments = functools.partial(
        pl.kernel,
        mesh=mesh,
        out_type=[
            jax.ShapeDtypeStruct((_NG, _QL), jnp.float32),
            jax.ShapeDtypeStruct((_NW, _QL), jnp.float32),
            jax.ShapeDtypeStruct((_NW, _QL), jnp.float32),
        ],
        scratch_types=[
            pltpu.VMEM((_XW,), jnp.float32),
            pltpu.VMEM((_QL,), jnp.float32),
            pltpu.VMEM((_QL,), jnp.float32),
            pltpu.VMEM((_QL,), jnp.float32),
            pltpu.VMEM((_FP * _L,), jnp.float32),
        ],
    )(_sc_moments_body)
    inter, s1p, s2p = moments(xg)

    alphav, cvec = pl.pallas_call(
        _finalize_kernel,
        in_specs=[
            pl.BlockSpec((_NW, _QL), lambda: (0, 0)),
            pl.BlockSpec((_NW, _QL), lambda: (0, 0)),
            pl.BlockSpec((1, _QL), lambda: (0, 0)),
            pl.BlockSpec((1, _QL), lambda: (0, 0)),
            pl.BlockSpec((1, _QL), lambda: (0, 0)),
        ],
        out_specs=[
            pl.BlockSpec((1, _QL), lambda: (0, 0)),
            pl.BlockSpec((1, _L), lambda: (0, 0)),
        ],
        out_shape=[
            jax.ShapeDtypeStruct((1, _QL), jnp.float32),
            jax.ShapeDtypeStruct((1, _L), jnp.float32),
        ],
    )(s1p, s2p, wm, gm, bm)

    out = functools.partial(
        pl.kernel,
        mesh=mesh,
        out_type=jax.ShapeDtypeStruct((_B,), jnp.float32),
        scratch_types=[
            pltpu.VMEM((_QL,), jnp.float32),
            pltpu.VMEM((_QL,), jnp.float32),
            pltpu.VMEM((_L,), jnp.float32),
            pltpu.VMEM((_L,), jnp.float32),
        ],
    )(_sc_out_body)(inter, alphav.reshape(_QL), cvec.reshape(_L))

    return out.reshape(_B, 1)
